# baseline (speedup 1.0000x reference)
import functools

import jax
import jax.numpy as jnp
from jax import lax
from jax.experimental import pallas as pl
from jax.experimental.pallas import tpu as pltpu
from jax.experimental.pallas import tpu_sc as plsc

WD = 0.0001
D = 64
CHUNK = 128

_SC_PARAMS = dict(
    compiler_params=pltpu.CompilerParams(needs_layout_passes=False,
                                         use_tc_tiling_on_sc=False),
)


def _mesh_info():
    info = plsc.get_sparse_core_info()
    return info.num_cores, info.num_subcores, info.num_lanes


def _sc_gather_u(u, W):
    B = u.shape[0]
    NC, NS, L = _mesh_info()
    NW = NC * NS
    BPW = B // NW
    NCHUNK = BPW // CHUNK
    mesh = plsc.VectorSubcoreMesh(core_axis_name="c", subcore_axis_name="s")

    @functools.partial(
        pl.kernel,
        out_type=jax.ShapeDtypeStruct((B, D), jnp.float32),
        mesh=mesh,
        scratch_types=[
            pltpu.VMEM((NCHUNK, CHUNK), jnp.int32),
            pltpu.VMEM((BPW, D), jnp.float32),
            pltpu.SemaphoreType.DMA,
        ],
        **_SC_PARAMS,
    )
    def a_kernel(u_hbm, W_hbm, rows_hbm, u_idx, u_rows, su):
        wid = lax.axis_index("s") * NC + lax.axis_index("c")
        base = wid * BPW
        for k in range(NCHUNK):
            pltpu.sync_copy(u_hbm.at[pl.ds(base + k * CHUNK, CHUNK)],
                            u_idx.at[k])
        copies = []
        for k in range(NCHUNK):
            dst = pl.ds(k * CHUNK, CHUNK)
            copies.append(
                pltpu.async_copy(W_hbm.at[u_idx.at[k]], u_rows.at[dst], su))
        for c in copies:
            c.wait()
        pltpu.sync_copy(u_rows, rows_hbm.at[pl.ds(base, BPW)])

    return a_kernel(u, W)


def _sc_gather_dot(i, j, H, u_rows_in):
    B = i.shape[0]
    NC, NS, L = _mesh_info()
    NW = NC * NS
    BPW = B // NW
    NCHUNK = BPW // CHUNK
    NGROUP = BPW // L
    mesh = plsc.VectorSubcoreMesh(core_axis_name="c", subcore_axis_name="s")

    @functools.partial(
        pl.kernel,
        out_type=[
            jax.ShapeDtypeStruct((B,), jnp.float32),
            jax.ShapeDtypeStruct((NW * L,), jnp.float32),
        ],
        mesh=mesh,
        scratch_types=[
            pltpu.VMEM((NCHUNK, CHUNK), jnp.int32),
            pltpu.VMEM((NCHUNK, CHUNK), jnp.int32),
            pltpu.VMEM((BPW, D), jnp.float32),
            pltpu.VMEM((BPW, D), jnp.float32),
            pltpu.VMEM((BPW, D), jnp.float32),
            pltpu.VMEM((BPW,), jnp.float32),
            pltpu.VMEM((L,), jnp.float32),
            pltpu.SemaphoreType.DMA,
            pltpu.SemaphoreType.DMA,
            pltpu.SemaphoreType.DMA,
        ],
        **_SC_PARAMS,
    )
    def b_kernel(i_hbm, j_hbm, H_hbm, urows_hbm, x_hbm, sq_hbm,
                 i_idx, j_idx, u_rows, i_rows, j_rows, x_v, sq_v,
                 su, si, sj):
        wid = lax.axis_index("s") * NC + lax.axis_index("c")
        base = wid * BPW

        cu = pltpu.async_copy(urows_hbm.at[pl.ds(base, BPW)], u_rows, su)
        for k in range(NCHUNK):
            pltpu.sync_copy(i_hbm.at[pl.ds(base + k * CHUNK, CHUNK)],
                            i_idx.at[k])
            pltpu.sync_copy(j_hbm.at[pl.ds(base + k * CHUNK, CHUNK)],
                            j_idx.at[k])
        copies = []
        for k in range(NCHUNK):
            dst = pl.ds(k * CHUNK, CHUNK)
            copies.append(
                pltpu.async_copy(H_hbm.at[i_idx.at[k]], i_rows.at[dst], si))
            copies.append(
                pltpu.async_copy(H_hbm.at[j_idx.at[k]], j_rows.at[dst], sj))
        cu.wait()
        for c in copies:
            c.wait()

        lanes = lax.iota(jnp.int32, L)

        def group_body(g, sq_acc):
            svec = jnp.zeros((L,), jnp.float32)
            for r in range(L):
                row = g * L + r
                acc = jnp.zeros((L,), jnp.float32)
                for c in range(D // L):
                    sl = pl.ds(c * L, L)
                    uv = u_rows[row, sl]
                    iv = i_rows[row, sl]
                    jv = j_rows[row, sl]
                    acc = acc + uv * (iv - jv)
                    sq_acc = sq_acc + (uv * uv + (iv * iv + jv * jv))
                s = jnp.sum(acc)
                svec = jnp.where(lanes == r, s, svec)
            x_v[pl.ds(g * L, L)] = svec
            return sq_acc

        sq_acc = lax.fori_loop(0, NGROUP, group_body,
                               jnp.zeros((L,), jnp.float32))
        sq_v[...] = sq_acc
        pltpu.sync_copy(x_v, x_hbm.at[pl.ds(base, BPW)])
        pltpu.sync_copy(sq_v, sq_hbm.at[pl.ds(wid * L, L)])

    return b_kernel(i, j, H, u_rows_in)


def _tc_finish(x2d, sq2d):
    def body(x_ref, sq_ref, o_ref):
        x = x_ref[...]
        ls = jnp.minimum(x, 0.0) - jnp.log1p(jnp.exp(-jnp.abs(x)))
        o_ref[0, 0] = WD * jnp.sum(sq_ref[...]) - jnp.sum(ls)

    return pl.pallas_call(
        body,
        out_shape=jax.ShapeDtypeStruct((1, 1), jnp.float32),
        out_specs=pl.BlockSpec(memory_space=pltpu.SMEM),
    )(x2d, sq2d)


def kernel(u, i, j, W, H):
    u = u.astype(jnp.int32)
    i = i.astype(jnp.int32)
    j = j.astype(jnp.int32)
    u_rows = _sc_gather_u(u, W)
    x, sq = _sc_gather_dot(i, j, H, u_rows)
    out = _tc_finish(x.reshape(128, -1), sq.reshape(4, -1))
    return out[0, 0]
